# baseline (speedup 1.0000x reference)
import functools
import math

import jax
import jax.numpy as jnp
from jax import lax
from jax.experimental import pallas as pl
from jax.experimental.pallas import tpu as pltpu
from jax.experimental.pallas import tpu_sc as plsc

N_NODES = 10000
K = 27
E_PER_K = 12000
E = K * E_PER_K
D = 128

SCALE = math.sqrt(2.0) / math.sqrt(float(K * D))

NC = 2
NS = 16
NW = NC * NS
B = 128
PH = 2
ACC_ROWS = 10240
ROWS_PER_TILE = ACC_ROWS // NS

K_A = 14
K_B = K - K_A
E_A = K_A * E_PER_K
E_B = K_B * E_PER_K


def _z_body(x_ref, w_ref, z_ref):
    z_ref[0] = jnp.dot(x_ref[...], w_ref[0] * SCALE,
                       preferred_element_type=jnp.float32)


def _compute_z(x, Wc, kk):
    return pl.pallas_call(
        _z_body,
        grid=(kk,),
        in_specs=[
            pl.BlockSpec((N_NODES, D), lambda k: (0, 0)),
            pl.BlockSpec((1, D, D), lambda k: (k, 0, 0)),
        ],
        out_specs=pl.BlockSpec((1, N_NODES, D), lambda k: (k, 0, 0)),
        out_shape=jax.ShapeDtypeStruct((kk, N_NODES, D), jnp.float32),
    )(x, Wc)


_mesh = plsc.VectorSubcoreMesh(core_axis_name="c", subcore_axis_name="s")


def _make_sc_scatter(nbp):
    nbpi = nbp + 1

    @functools.partial(
        pl.kernel,
        mesh=_mesh,
        out_type=jax.ShapeDtypeStruct((NC, ACC_ROWS, D), jnp.float32),
        scratch_types=[
            pltpu.VMEM((nbpi, B), jnp.int32),
            pltpu.VMEM((nbp, B), jnp.int32),
            pltpu.VMEM((B, D), jnp.float32),
            pltpu.VMEM((B, D), jnp.float32),
            pltpu.VMEM_SHARED((ACC_ROWS, D), jnp.float32),
            pltpu.SemaphoreType.DMA,
            pltpu.SemaphoreType.DMA,
        ],
    )
    def _sc_scatter(z_hbm, gidx_hbm, dst_hbm, out_hbm, idx_v, dst_v, rows0,
                    rows1, acc_sh, sem0, sem1):
        c = lax.axis_index("c")
        s = lax.axis_index("s")
        wid = c * NS + s

        def _zero_row(r, carry):
            for cc in range(D // 16):
                rows0[r, pl.ds(cc * 16, 16)] = jnp.zeros((16,), jnp.float32)
            return carry

        lax.fori_loop(0, B, _zero_row, 0)

        def _zero_chunk(i, carry):
            pltpu.sync_copy(rows0,
                            acc_sh.at[pl.ds(s * ROWS_PER_TILE + i * B, B)])
            return carry

        lax.fori_loop(0, ROWS_PER_TILE // B, _zero_chunk, 0)
        plsc.subcore_barrier()

        def _fire(j, buf, sem):
            pltpu.async_copy(z_hbm.at[idx_v.at[j]], buf, sem)

        def _wait(j, buf, sem):
            pltpu.make_async_copy(z_hbm.at[idx_v.at[j]], buf, sem).wait()

        def _scat(j, buf):
            pltpu.sync_copy(buf, acc_sh.at[dst_v.at[j]], add=True)

        for p in range(PH):
            pltpu.sync_copy(gidx_hbm.at[wid, p], idx_v)
            pltpu.sync_copy(dst_hbm.at[wid, p], dst_v)
            _fire(0, rows0, sem0)

            def _pair(g, carry):
                j = 2 * g
                _fire(j + 1, rows1, sem1)
                _wait(j, rows0, sem0)
                _scat(j, rows0)
                _fire(j + 2, rows0, sem0)
                _wait(j + 1, rows1, sem1)
                _scat(j + 1, rows1)
                return carry

            lax.fori_loop(0, nbp // 2, _pair, 0)
            if nbp % 2:
                _wait(nbp - 1, rows0, sem0)
                _scat(nbp - 1, rows0)
            else:
                _wait(nbp, rows0, sem0)
        plsc.subcore_barrier()

        pltpu.sync_copy(acc_sh.at[pl.ds(s * ROWS_PER_TILE, ROWS_PER_TILE)],
                        out_hbm.at[c, pl.ds(s * ROWS_PER_TILE, ROWS_PER_TILE)])

    return _sc_scatter


def _merge_body(pa_ref, pb_ref, b_ref, o_ref):
    o_ref[...] = (pa_ref[0] + pa_ref[1]) + (pb_ref[0] + pb_ref[1]) + b_ref[...]


def _merge(pa, pb, bias2d):
    blk = 2000
    return pl.pallas_call(
        _merge_body,
        grid=(N_NODES // blk,),
        in_specs=[
            pl.BlockSpec((NC, blk, D), lambda i: (0, i, 0)),
            pl.BlockSpec((NC, blk, D), lambda i: (0, i, 0)),
            pl.BlockSpec((1, D), lambda i: (0, 0)),
        ],
        out_specs=pl.BlockSpec((blk, D), lambda i: (i, 0)),
        out_shape=jax.ShapeDtypeStruct((N_NODES, D), jnp.float32),
    )(pa, pb, bias2d)


def _prep_indices(gidx, dst, n_edges, nbp):
    e_pad = NW * PH * nbp * B
    npad = e_pad - n_edges
    pad_g = jnp.arange(npad, dtype=jnp.int32) % N_NODES
    pad_d = N_NODES + jnp.arange(npad, dtype=jnp.int32) % (ACC_ROWS - N_NODES)
    gidx_p = jnp.concatenate([gidx, pad_g]).reshape(NW, PH, nbp, B)
    dst_p = jnp.concatenate([dst, pad_d]).reshape(NW, PH, nbp, B)
    dummy = (jnp.arange(NW * PH * B, dtype=jnp.int32)
             .reshape(NW, PH, 1, B) % N_NODES)
    gidx_p = jnp.concatenate([gidx_p, dummy], axis=2)
    return gidx_p, dst_p


_SLOT = NW * PH * B
NBP_A = -(-E_A // _SLOT)
NBP_B = -(-E_B // _SLOT)
_sc_scatter_a = _make_sc_scatter(NBP_A)
_sc_scatter_b = _make_sc_scatter(NBP_B)


def kernel(x, edge_index, W, bias):
    src = edge_index[0]
    dst = edge_index[1]
    k_of_e = jnp.arange(E, dtype=jnp.int32) // E_PER_K
    gidx_a = src[:E_A] + k_of_e[:E_A] * N_NODES
    gidx_b = src[E_A:] + (k_of_e[E_A:] - K_A) * N_NODES
    ga, da = _prep_indices(gidx_a, dst[:E_A], E_A, NBP_A)
    gb, db = _prep_indices(gidx_b, dst[E_A:], E_B, NBP_B)

    z_a = _compute_z(x, W[:K_A], K_A).reshape(K_A * N_NODES, D)
    z_b = _compute_z(x, W[K_A:], K_B).reshape(K_B * N_NODES, D)
    pa = _sc_scatter_a(z_a, ga, da)
    pb = _sc_scatter_b(z_b, gb, db)
    return _merge(pa, pb, bias.reshape(1, D))


# speedup vs baseline: 1.0577x; 1.0577x over previous
import functools
import math

import jax
import jax.numpy as jnp
from jax import lax
from jax.experimental import pallas as pl
from jax.experimental.pallas import tpu as pltpu
from jax.experimental.pallas import tpu_sc as plsc

N_NODES = 10000
K = 27
E_PER_K = 12000
E = K * E_PER_K
D = 128

SCALE = math.sqrt(2.0) / math.sqrt(float(K * D))

NC = 2
NS = 16
NW = NC * NS
B = 128
PH = 2
ACC_ROWS = 10240
ROWS_PER_TILE = ACC_ROWS // NS

K_A = 14
K_B = K - K_A
E_A = K_A * E_PER_K
E_B = K_B * E_PER_K


def _z_body(x_ref, w_ref, z_ref):
    z_ref[0] = jnp.dot(x_ref[...], w_ref[0] * SCALE,
                       preferred_element_type=jnp.float32)


def _compute_z(x, Wc, kk):
    return pl.pallas_call(
        _z_body,
        grid=(kk,),
        in_specs=[
            pl.BlockSpec((N_NODES, D), lambda k: (0, 0)),
            pl.BlockSpec((1, D, D), lambda k: (k, 0, 0)),
        ],
        out_specs=pl.BlockSpec((1, N_NODES, D), lambda k: (k, 0, 0)),
        out_shape=jax.ShapeDtypeStruct((kk, N_NODES, D), jnp.float32),
    )(x, Wc)


_mesh = plsc.VectorSubcoreMesh(core_axis_name="c", subcore_axis_name="s")


def _make_sc_scatter(nbp):
    nbpi = nbp + 1

    @functools.partial(
        pl.kernel,
        mesh=_mesh,
        out_type=jax.ShapeDtypeStruct((NC, ACC_ROWS, D), jnp.float32),
        scratch_types=[
            pltpu.VMEM((nbpi, B), jnp.int32),
            pltpu.VMEM((nbp, B), jnp.int32),
            pltpu.VMEM((B, D), jnp.float32),
            pltpu.VMEM((B, D), jnp.float32),
            pltpu.VMEM_SHARED((ACC_ROWS, D), jnp.float32),
            pltpu.SemaphoreType.DMA,
            pltpu.SemaphoreType.DMA,
        ],
    )
    def _sc_scatter(z_hbm, gidx_hbm, dst_hbm, out_hbm, idx_v, dst_v, rows0,
                    rows1, acc_sh, sem0, sem1):
        c = lax.axis_index("c")
        s = lax.axis_index("s")
        wid = c * NS + s

        def _zero_row(r, carry):
            for cc in range(D // 16):
                rows0[r, pl.ds(cc * 16, 16)] = jnp.zeros((16,), jnp.float32)
            return carry

        lax.fori_loop(0, B, _zero_row, 0)

        def _zero_chunk(i, carry):
            pltpu.sync_copy(rows0,
                            acc_sh.at[pl.ds(s * ROWS_PER_TILE + i * B, B)])
            return carry

        lax.fori_loop(0, ROWS_PER_TILE // B, _zero_chunk, 0)
        plsc.subcore_barrier()

        def _fire(j, buf, sem):
            pltpu.async_copy(z_hbm.at[idx_v.at[j]], buf, sem)

        def _wait(j, buf, sem):
            pltpu.make_async_copy(z_hbm.at[idx_v.at[j]], buf, sem).wait()

        def _scat(j, buf):
            pltpu.sync_copy(buf, acc_sh.at[dst_v.at[j]], add=True)

        for p in range(PH):
            pltpu.sync_copy(gidx_hbm.at[wid, p], idx_v)
            pltpu.sync_copy(dst_hbm.at[wid, p], dst_v)
            _fire(0, rows0, sem0)

            def _pair(g, carry):
                j = 2 * g
                _fire(j + 1, rows1, sem1)
                _wait(j, rows0, sem0)
                _scat(j, rows0)
                _fire(j + 2, rows0, sem0)
                _wait(j + 1, rows1, sem1)
                _scat(j + 1, rows1)
                return carry

            lax.fori_loop(0, nbp // 2, _pair, 0)
            if nbp % 2:
                _wait(nbp - 1, rows0, sem0)
                _scat(nbp - 1, rows0)
            else:
                _wait(nbp, rows0, sem0)
        plsc.subcore_barrier()

        pltpu.sync_copy(acc_sh.at[pl.ds(s * ROWS_PER_TILE, ROWS_PER_TILE)],
                        out_hbm.at[c, pl.ds(s * ROWS_PER_TILE, ROWS_PER_TILE)])

    return _sc_scatter


def _merge_body(pa_ref, pb_ref, b_ref, o_ref):
    o_ref[...] = (pa_ref[0] + pa_ref[1]) + (pb_ref[0] + pb_ref[1]) + b_ref[...]


def _merge(pa, pb, bias2d):
    blk = 2000
    return pl.pallas_call(
        _merge_body,
        grid=(N_NODES // blk,),
        in_specs=[
            pl.BlockSpec((NC, blk, D), lambda i: (0, i, 0)),
            pl.BlockSpec((NC, blk, D), lambda i: (0, i, 0)),
            pl.BlockSpec((1, D), lambda i: (0, 0)),
        ],
        out_specs=pl.BlockSpec((blk, D), lambda i: (i, 0)),
        out_shape=jax.ShapeDtypeStruct((N_NODES, D), jnp.float32),
    )(pa, pb, bias2d)


def _prep_indices(gidx, dst, n_edges, nbp):
    e_pad = NW * PH * nbp * B
    npad = e_pad - n_edges
    pad_g = jnp.arange(npad, dtype=jnp.int32) % N_NODES
    pad_d = N_NODES + jnp.arange(npad, dtype=jnp.int32) % (ACC_ROWS - N_NODES)
    gidx_p = jnp.concatenate([gidx, pad_g]).reshape(NW, PH, nbp, B)
    dst_p = jnp.concatenate([dst, pad_d]).reshape(NW, PH, nbp, B)
    dummy = (jnp.arange(NW * PH * B, dtype=jnp.int32)
             .reshape(NW, PH, 1, B) % N_NODES)
    gidx_p = jnp.concatenate([gidx_p, dummy], axis=2)
    return gidx_p, dst_p


_SLOT = NW * PH * B
NBP_F = -(-E // _SLOT)
_sc_scatter_full = _make_sc_scatter(NBP_F)


def _merge1_body(pa_ref, b_ref, o_ref):
    o_ref[...] = pa_ref[0] + pa_ref[1] + b_ref[...]


def _merge1(pa, bias2d):
    blk = 2000
    return pl.pallas_call(
        _merge1_body,
        grid=(N_NODES // blk,),
        in_specs=[
            pl.BlockSpec((NC, blk, D), lambda i: (0, i, 0)),
            pl.BlockSpec((1, D), lambda i: (0, 0)),
        ],
        out_specs=pl.BlockSpec((blk, D), lambda i: (i, 0)),
        out_shape=jax.ShapeDtypeStruct((N_NODES, D), jnp.float32),
    )(pa, bias2d)


def kernel(x, edge_index, W, bias):
    src = edge_index[0]
    dst = edge_index[1]
    k_of_e = jnp.arange(E, dtype=jnp.int32) // E_PER_K
    gidx = src + k_of_e * N_NODES
    ga, da = _prep_indices(gidx, dst, E, NBP_F)

    z = _compute_z(x, W, K).reshape(K * N_NODES, D)
    pa = _sc_scatter_full(z, ga, da)
    return _merge1(pa, bias.reshape(1, D))


# speedup vs baseline: 1.0676x; 1.0094x over previous
import functools
import math

import jax
import jax.numpy as jnp
from jax import lax
from jax.experimental import pallas as pl
from jax.experimental.pallas import tpu as pltpu
from jax.experimental.pallas import tpu_sc as plsc

N_NODES = 10000
K = 27
E_PER_K = 12000
E = K * E_PER_K
D = 128

SCALE = math.sqrt(2.0) / math.sqrt(float(K * D))

NC = 2
NS = 16
NW = NC * NS
B = 128
PH = 2
ACC_ROWS = 10240
ROWS_PER_TILE = ACC_ROWS // NS

K_A = 14
K_B = K - K_A
E_A = K_A * E_PER_K
E_B = K_B * E_PER_K


def _z_body(x_ref, w_ref, z_ref):
    z_ref[0] = jnp.dot(x_ref[...], w_ref[0] * SCALE,
                       preferred_element_type=jnp.float32)


def _compute_z(x, Wc, kk):
    return pl.pallas_call(
        _z_body,
        grid=(kk,),
        in_specs=[
            pl.BlockSpec((N_NODES, D), lambda k: (0, 0)),
            pl.BlockSpec((1, D, D), lambda k: (k, 0, 0)),
        ],
        out_specs=pl.BlockSpec((1, N_NODES, D), lambda k: (k, 0, 0)),
        out_shape=jax.ShapeDtypeStruct((kk, N_NODES, D), jnp.float32),
    )(x, Wc)


_mesh = plsc.VectorSubcoreMesh(core_axis_name="c", subcore_axis_name="s")


def _make_sc_scatter(nbp):
    nbpi = nbp + 1

    @functools.partial(
        pl.kernel,
        mesh=_mesh,
        out_type=jax.ShapeDtypeStruct((NC, ACC_ROWS, D), jnp.float32),
        scratch_types=[
            pltpu.VMEM((nbpi, B), jnp.int32),
            pltpu.VMEM((nbp, B), jnp.int32),
            pltpu.VMEM((B, D), jnp.float32),
            pltpu.VMEM((B, D), jnp.float32),
            pltpu.VMEM_SHARED((ACC_ROWS, D), jnp.float32),
            pltpu.SemaphoreType.DMA,
            pltpu.SemaphoreType.DMA,
        ],
    )
    def _sc_scatter(z_hbm, gidx_hbm, dst_hbm, out_hbm, idx_v, dst_v, rows0,
                    rows1, acc_sh, sem0, sem1):
        c = lax.axis_index("c")
        s = lax.axis_index("s")
        wid = c * NS + s

        pltpu.async_copy(gidx_hbm.at[wid, 0], idx_v, sem1)
        pltpu.async_copy(dst_hbm.at[wid, 0], dst_v, sem1)

        def _zero_row(r, carry):
            for cc in range(D // 16):
                rows0[r, pl.ds(cc * 16, 16)] = jnp.zeros((16,), jnp.float32)
            return carry

        lax.fori_loop(0, B, _zero_row, 0)
        for i in range(ROWS_PER_TILE // B):
            pltpu.async_copy(
                rows0, acc_sh.at[pl.ds(s * ROWS_PER_TILE + i * B, B)], sem0)
        for i in range(ROWS_PER_TILE // B):
            pltpu.make_async_copy(
                rows0, acc_sh.at[pl.ds(s * ROWS_PER_TILE + i * B, B)],
                sem0).wait()
        pltpu.make_async_copy(gidx_hbm.at[wid, 0], idx_v, sem1).wait()
        pltpu.make_async_copy(dst_hbm.at[wid, 0], dst_v, sem1).wait()
        plsc.subcore_barrier()

        def _fire(j, buf, sem):
            pltpu.async_copy(z_hbm.at[idx_v.at[j]], buf, sem)

        def _wait(j, buf, sem):
            pltpu.make_async_copy(z_hbm.at[idx_v.at[j]], buf, sem).wait()

        def _scat(j, buf):
            pltpu.sync_copy(buf, acc_sh.at[dst_v.at[j]], add=True)

        for p in range(PH):
            if p > 0:
                pltpu.sync_copy(gidx_hbm.at[wid, p], idx_v)
                pltpu.sync_copy(dst_hbm.at[wid, p], dst_v)
            _fire(0, rows0, sem0)

            def _pair(g, carry):
                j = 2 * g
                _fire(j + 1, rows1, sem1)
                _wait(j, rows0, sem0)
                _scat(j, rows0)
                _fire(j + 2, rows0, sem0)
                _wait(j + 1, rows1, sem1)
                _scat(j + 1, rows1)
                return carry

            lax.fori_loop(0, nbp // 2, _pair, 0)
            if nbp % 2:
                _wait(nbp - 1, rows0, sem0)
                _scat(nbp - 1, rows0)
            else:
                _wait(nbp, rows0, sem0)
        plsc.subcore_barrier()

        pltpu.sync_copy(acc_sh.at[pl.ds(s * ROWS_PER_TILE, ROWS_PER_TILE)],
                        out_hbm.at[c, pl.ds(s * ROWS_PER_TILE, ROWS_PER_TILE)])

    return _sc_scatter


def _merge_body(pa_ref, pb_ref, b_ref, o_ref):
    o_ref[...] = (pa_ref[0] + pa_ref[1]) + (pb_ref[0] + pb_ref[1]) + b_ref[...]


def _merge(pa, pb, bias2d):
    blk = 2000
    return pl.pallas_call(
        _merge_body,
        grid=(N_NODES // blk,),
        in_specs=[
            pl.BlockSpec((NC, blk, D), lambda i: (0, i, 0)),
            pl.BlockSpec((NC, blk, D), lambda i: (0, i, 0)),
            pl.BlockSpec((1, D), lambda i: (0, 0)),
        ],
        out_specs=pl.BlockSpec((blk, D), lambda i: (i, 0)),
        out_shape=jax.ShapeDtypeStruct((N_NODES, D), jnp.float32),
    )(pa, pb, bias2d)


def _prep_indices(gidx, dst, n_edges, nbp):
    e_pad = NW * PH * nbp * B
    npad = e_pad - n_edges
    pad_g = jnp.arange(npad, dtype=jnp.int32) % N_NODES
    pad_d = N_NODES + jnp.arange(npad, dtype=jnp.int32) % (ACC_ROWS - N_NODES)
    gidx_p = jnp.concatenate([gidx, pad_g]).reshape(NW, PH, nbp, B)
    dst_p = jnp.concatenate([dst, pad_d]).reshape(NW, PH, nbp, B)
    dummy = (jnp.arange(NW * PH * B, dtype=jnp.int32)
             .reshape(NW, PH, 1, B) % N_NODES)
    gidx_p = jnp.concatenate([gidx_p, dummy], axis=2)
    return gidx_p, dst_p


_SLOT = NW * PH * B
NBP_F = -(-E // _SLOT)
_sc_scatter_full = _make_sc_scatter(NBP_F)


def _merge1_body(pa_ref, b_ref, o_ref):
    o_ref[...] = pa_ref[0] + pa_ref[1] + b_ref[...]


def _merge1(pa, bias2d):
    blk = 2000
    return pl.pallas_call(
        _merge1_body,
        grid=(N_NODES // blk,),
        in_specs=[
            pl.BlockSpec((NC, blk, D), lambda i: (0, i, 0)),
            pl.BlockSpec((1, D), lambda i: (0, 0)),
        ],
        out_specs=pl.BlockSpec((blk, D), lambda i: (i, 0)),
        out_shape=jax.ShapeDtypeStruct((N_NODES, D), jnp.float32),
    )(pa, bias2d)


def kernel(x, edge_index, W, bias):
    src = edge_index[0]
    dst = edge_index[1]
    k_of_e = jnp.arange(E, dtype=jnp.int32) // E_PER_K
    gidx = src + k_of_e * N_NODES
    ga, da = _prep_indices(gidx, dst, E, NBP_F)

    z = _compute_z(x, W, K).reshape(K * N_NODES, D)
    pa = _sc_scatter_full(z, ga, da)
    return _merge1(pa, bias.reshape(1, D))


# speedup vs baseline: 1.0696x; 1.0019x over previous
import functools
import math

import numpy as np

import jax
import jax.numpy as jnp
from jax import lax
from jax.experimental import pallas as pl
from jax.experimental.pallas import tpu as pltpu
from jax.experimental.pallas import tpu_sc as plsc

N_NODES = 10000
K = 27
E_PER_K = 12000
E = K * E_PER_K
D = 128

SCALE = math.sqrt(2.0) / math.sqrt(float(K * D))

NC = 2
NS = 16
NW = NC * NS
B = 128
PH = 2
ACC_ROWS = 10240
ROWS_PER_TILE = ACC_ROWS // NS

K_A = 14
K_B = K - K_A
E_A = K_A * E_PER_K
E_B = K_B * E_PER_K


def _z_body(x_ref, w_ref, z_ref):
    z_ref[0] = jnp.dot(x_ref[...], w_ref[0] * SCALE,
                       preferred_element_type=jnp.float32)


def _compute_z(x, Wc, kk):
    return pl.pallas_call(
        _z_body,
        grid=(kk,),
        in_specs=[
            pl.BlockSpec((N_NODES, D), lambda k: (0, 0)),
            pl.BlockSpec((1, D, D), lambda k: (k, 0, 0)),
        ],
        out_specs=pl.BlockSpec((1, N_NODES, D), lambda k: (k, 0, 0)),
        out_shape=jax.ShapeDtypeStruct((kk, N_NODES, D), jnp.float32),
    )(x, Wc)


_mesh = plsc.VectorSubcoreMesh(core_axis_name="c", subcore_axis_name="s")


def _make_sc_scatter(nbp):
    nbpi = nbp + 1

    @functools.partial(
        pl.kernel,
        mesh=_mesh,
        out_type=jax.ShapeDtypeStruct((NC, ACC_ROWS, D), jnp.float32),
        scratch_types=[
            pltpu.VMEM((nbpi, B), jnp.int32),
            pltpu.VMEM((nbp, B), jnp.int32),
            pltpu.VMEM((B, D), jnp.float32),
            pltpu.VMEM((B, D), jnp.float32),
            pltpu.VMEM_SHARED((ACC_ROWS, D), jnp.float32),
            pltpu.SemaphoreType.DMA,
            pltpu.SemaphoreType.DMA,
        ],
    )
    def _sc_scatter(z_hbm, gidx_hbm, dst_hbm, out_hbm, idx_v, dst_v, rows0,
                    rows1, acc_sh, sem0, sem1):
        c = lax.axis_index("c")
        s = lax.axis_index("s")
        wid = c * NS + s

        pltpu.async_copy(gidx_hbm.at[wid, 0], idx_v, sem1)
        pltpu.async_copy(dst_hbm.at[wid, 0], dst_v, sem1)

        def _zero_row(r, carry):
            for cc in range(D // 16):
                rows0[r, pl.ds(cc * 16, 16)] = jnp.zeros((16,), jnp.float32)
            return carry

        lax.fori_loop(0, B, _zero_row, 0)
        for i in range(ROWS_PER_TILE // B):
            pltpu.async_copy(
                rows0, acc_sh.at[pl.ds(s * ROWS_PER_TILE + i * B, B)], sem0)
        for i in range(ROWS_PER_TILE // B):
            pltpu.make_async_copy(
                rows0, acc_sh.at[pl.ds(s * ROWS_PER_TILE + i * B, B)],
                sem0).wait()
        pltpu.make_async_copy(gidx_hbm.at[wid, 0], idx_v, sem1).wait()
        pltpu.make_async_copy(dst_hbm.at[wid, 0], dst_v, sem1).wait()
        plsc.subcore_barrier()

        def _fire(j, buf, sem):
            pltpu.async_copy(z_hbm.at[idx_v.at[j]], buf, sem)

        def _wait(j, buf, sem):
            pltpu.make_async_copy(z_hbm.at[idx_v.at[j]], buf, sem).wait()

        def _scat(j, buf):
            pltpu.sync_copy(buf, acc_sh.at[dst_v.at[j]], add=True)

        for p in range(PH):
            if p > 0:
                pltpu.sync_copy(gidx_hbm.at[wid, p], idx_v)
                pltpu.sync_copy(dst_hbm.at[wid, p], dst_v)
            _fire(0, rows0, sem0)

            def _pair(g, carry):
                j = 2 * g
                _fire(j + 1, rows1, sem1)
                _wait(j, rows0, sem0)
                _scat(j, rows0)
                _fire(j + 2, rows0, sem0)
                _wait(j + 1, rows1, sem1)
                _scat(j + 1, rows1)
                return carry

            lax.fori_loop(0, nbp // 2, _pair, 0)
            if nbp % 2:
                _wait(nbp - 1, rows0, sem0)
                _scat(nbp - 1, rows0)
            else:
                _wait(nbp, rows0, sem0)
        plsc.subcore_barrier()

        pltpu.sync_copy(acc_sh.at[pl.ds(s * ROWS_PER_TILE, ROWS_PER_TILE)],
                        out_hbm.at[c, pl.ds(s * ROWS_PER_TILE, ROWS_PER_TILE)])

    return _sc_scatter


def _merge_body(pa_ref, pb_ref, b_ref, o_ref):
    o_ref[...] = (pa_ref[0] + pa_ref[1]) + (pb_ref[0] + pb_ref[1]) + b_ref[...]


def _merge(pa, pb, bias2d):
    blk = 2000
    return pl.pallas_call(
        _merge_body,
        grid=(N_NODES // blk,),
        in_specs=[
            pl.BlockSpec((NC, blk, D), lambda i: (0, i, 0)),
            pl.BlockSpec((NC, blk, D), lambda i: (0, i, 0)),
            pl.BlockSpec((1, D), lambda i: (0, 0)),
        ],
        out_specs=pl.BlockSpec((blk, D), lambda i: (i, 0)),
        out_shape=jax.ShapeDtypeStruct((N_NODES, D), jnp.float32),
    )(pa, pb, bias2d)


def _prep_indices(gidx, dst, n_edges, nbp):
    e_pad = NW * PH * nbp * B
    npad = e_pad - n_edges
    pad_g = np.arange(npad, dtype=np.int32) % N_NODES
    pad_d = N_NODES + np.arange(npad, dtype=np.int32) % (ACC_ROWS - N_NODES)
    gidx_p = jnp.concatenate([gidx, jnp.asarray(pad_g)]).reshape(
        NW, PH, nbp, B)
    dst_p = jnp.concatenate([dst, jnp.asarray(pad_d)]).reshape(
        NW, PH, nbp, B)
    dummy = jnp.asarray(np.arange(NW * PH * B, dtype=np.int32)
                        .reshape(NW, PH, 1, B) % N_NODES)
    gidx_p = jnp.concatenate([gidx_p, dummy], axis=2)
    return gidx_p, dst_p


_SLOT = NW * PH * B
NBP_F = -(-E // _SLOT)
_sc_scatter_full = _make_sc_scatter(NBP_F)


def _merge1_body(pa_ref, b_ref, o_ref):
    o_ref[...] = pa_ref[0] + pa_ref[1] + b_ref[...]


def _merge1(pa, bias2d):
    blk = 2000
    return pl.pallas_call(
        _merge1_body,
        grid=(N_NODES // blk,),
        in_specs=[
            pl.BlockSpec((NC, blk, D), lambda i: (0, i, 0)),
            pl.BlockSpec((1, D), lambda i: (0, 0)),
        ],
        out_specs=pl.BlockSpec((blk, D), lambda i: (i, 0)),
        out_shape=jax.ShapeDtypeStruct((N_NODES, D), jnp.float32),
    )(pa, bias2d)


_KOFF = jnp.asarray((np.arange(E, dtype=np.int32) // E_PER_K) * N_NODES)


def kernel(x, edge_index, W, bias):
    src = edge_index[0]
    dst = edge_index[1]
    gidx = src + _KOFF
    ga, da = _prep_indices(gidx, dst, E, NBP_F)

    z = _compute_z(x, W, K).reshape(K * N_NODES, D)
    pa = _sc_scatter_full(z, ga, da)
    return _merge1(pa, bias.reshape(1, D))


# speedup vs baseline: 1.0705x; 1.0008x over previous
import functools
import math

import numpy as np

import jax
import jax.numpy as jnp
from jax import lax
from jax.experimental import pallas as pl
from jax.experimental.pallas import tpu as pltpu
from jax.experimental.pallas import tpu_sc as plsc

N_NODES = 10000
K = 27
E_PER_K = 12000
E = K * E_PER_K
D = 128

SCALE = math.sqrt(2.0) / math.sqrt(float(K * D))

NC = 2
NS = 16
NW = NC * NS
B = 128
PH = 2
ACC_ROWS = 10240
ROWS_PER_TILE = ACC_ROWS // NS

def _z_body(x_ref, w_ref, z_ref):
    z_ref[0] = jnp.dot(x_ref[...], w_ref[0] * SCALE,
                       preferred_element_type=jnp.float32)


def _compute_z(x, Wc, kk):
    return pl.pallas_call(
        _z_body,
        grid=(kk,),
        in_specs=[
            pl.BlockSpec((N_NODES, D), lambda k: (0, 0)),
            pl.BlockSpec((1, D, D), lambda k: (k, 0, 0)),
        ],
        out_specs=pl.BlockSpec((1, N_NODES, D), lambda k: (k, 0, 0)),
        out_shape=jax.ShapeDtypeStruct((kk, N_NODES, D), jnp.float32),
    )(x, Wc)


_mesh = plsc.VectorSubcoreMesh(core_axis_name="c", subcore_axis_name="s")


def _make_sc_scatter(nbp):
    nbpi = nbp + 1

    @functools.partial(
        pl.kernel,
        mesh=_mesh,
        out_type=jax.ShapeDtypeStruct((NC, ACC_ROWS, D), jnp.float32),
        scratch_types=[
            pltpu.VMEM((nbpi, B), jnp.int32),
            pltpu.VMEM((nbp, B), jnp.int32),
            pltpu.VMEM((B, D), jnp.float32),
            pltpu.VMEM((B, D), jnp.float32),
            pltpu.VMEM_SHARED((ACC_ROWS, D), jnp.float32),
            pltpu.SemaphoreType.DMA,
            pltpu.SemaphoreType.DMA,
        ],
    )
    def _sc_scatter(z_hbm, gidx_hbm, dst_hbm, out_hbm, idx_v, dst_v, rows0,
                    rows1, acc_sh, sem0, sem1):
        c = lax.axis_index("c")
        s = lax.axis_index("s")
        wid = c * NS + s

        pltpu.async_copy(gidx_hbm.at[wid, 0], idx_v, sem1)
        pltpu.async_copy(dst_hbm.at[wid, 0], dst_v, sem1)

        def _zero_row(r, carry):
            for cc in range(D // 16):
                rows0[r, pl.ds(cc * 16, 16)] = jnp.zeros((16,), jnp.float32)
            return carry

        lax.fori_loop(0, B, _zero_row, 0)
        for i in range(ROWS_PER_TILE // B):
            pltpu.async_copy(
                rows0, acc_sh.at[pl.ds(s * ROWS_PER_TILE + i * B, B)], sem0)
        for i in range(ROWS_PER_TILE // B):
            pltpu.make_async_copy(
                rows0, acc_sh.at[pl.ds(s * ROWS_PER_TILE + i * B, B)],
                sem0).wait()
        pltpu.make_async_copy(gidx_hbm.at[wid, 0], idx_v, sem1).wait()
        pltpu.make_async_copy(dst_hbm.at[wid, 0], dst_v, sem1).wait()
        plsc.subcore_barrier()

        def _fire(j, buf, sem):
            pltpu.async_copy(z_hbm.at[idx_v.at[j]], buf, sem)

        def _wait(j, buf, sem):
            pltpu.make_async_copy(z_hbm.at[idx_v.at[j]], buf, sem).wait()

        def _scat(j, buf):
            pltpu.sync_copy(buf, acc_sh.at[dst_v.at[j]], add=True)

        for p in range(PH):
            if p > 0:
                pltpu.sync_copy(gidx_hbm.at[wid, p], idx_v)
                pltpu.sync_copy(dst_hbm.at[wid, p], dst_v)
            _fire(0, rows0, sem0)

            def _pair(g, carry):
                j = 2 * g
                _fire(j + 1, rows1, sem1)
                _wait(j, rows0, sem0)
                _scat(j, rows0)
                _fire(j + 2, rows0, sem0)
                _wait(j + 1, rows1, sem1)
                _scat(j + 1, rows1)
                return carry

            lax.fori_loop(0, nbp // 2, _pair, 0)
            if nbp % 2:
                _wait(nbp - 1, rows0, sem0)
                _scat(nbp - 1, rows0)
            else:
                _wait(nbp, rows0, sem0)
        plsc.subcore_barrier()

        pltpu.sync_copy(acc_sh.at[pl.ds(s * ROWS_PER_TILE, ROWS_PER_TILE)],
                        out_hbm.at[c, pl.ds(s * ROWS_PER_TILE, ROWS_PER_TILE)])

    return _sc_scatter


def _prep_indices(gidx, dst, n_edges, nbp):
    e_pad = NW * PH * nbp * B
    npad = e_pad - n_edges
    pad_g = np.arange(npad, dtype=np.int32) % N_NODES
    pad_d = N_NODES + np.arange(npad, dtype=np.int32) % (ACC_ROWS - N_NODES)
    gidx_p = jnp.concatenate([gidx, jnp.asarray(pad_g)]).reshape(
        NW, PH, nbp, B)
    dst_p = jnp.concatenate([dst, jnp.asarray(pad_d)]).reshape(
        NW, PH, nbp, B)
    dummy = jnp.asarray(np.arange(NW * PH * B, dtype=np.int32)
                        .reshape(NW, PH, 1, B) % N_NODES)
    gidx_p = jnp.concatenate([gidx_p, dummy], axis=2)
    return gidx_p, dst_p


_SLOT = NW * PH * B
NBP_F = -(-E // _SLOT)
_sc_scatter_full = _make_sc_scatter(NBP_F)


def _merge1_body(pa_ref, b_ref, o_ref):
    o_ref[...] = pa_ref[0] + pa_ref[1] + b_ref[...]


def _merge1(pa, bias2d):
    blk = 2000
    return pl.pallas_call(
        _merge1_body,
        grid=(N_NODES // blk,),
        in_specs=[
            pl.BlockSpec((NC, blk, D), lambda i: (0, i, 0)),
            pl.BlockSpec((1, D), lambda i: (0, 0)),
        ],
        out_specs=pl.BlockSpec((blk, D), lambda i: (i, 0)),
        out_shape=jax.ShapeDtypeStruct((N_NODES, D), jnp.float32),
    )(pa, bias2d)


_KOFF = jnp.asarray((np.arange(E, dtype=np.int32) // E_PER_K) * N_NODES)


def kernel(x, edge_index, W, bias):
    src = edge_index[0]
    dst = edge_index[1]
    gidx = src + _KOFF
    ga, da = _prep_indices(gidx, dst, E, NBP_F)

    z = _compute_z(x, W, K).reshape(K * N_NODES, D)
    pa = _sc_scatter_full(z, ga, da)
    return _merge1(pa, bias.reshape(1, D))
